# baseline (speedup 1.0000x reference)
import functools

import jax
import jax.numpy as jnp
from jax import lax
from jax.experimental import pallas as pl
from jax.experimental.pallas import tpu as pltpu
from jax.experimental.pallas import tpu_sc as plsc

N = 10000
T = 8
F_IN = 128
H1 = 8
H2 = 16
D1 = T * H1
D2 = T * H2
NCORE = 2
NSUB = 16
NW = NCORE * NSUB
CHUNK = 128
NCHUNK = 80
EPW = NCHUNK * CHUNK
HALF = NCHUNK // 2
NLOOK = 2
SLAB = HALF + NLOOK
NPAD = 10016
RPW = NPAD // NSUB
BLK = 1000
NBLK = N // BLK

def _mesh():
    return plsc.VectorSubcoreMesh(
        core_axis_name="c", subcore_axis_name="s",
        num_cores=NCORE, num_subcores=NSUB,
    )


_DEG_K = 8


def _deg_body(srcd_hbm, dst_hbm, ones_hbm, zeros_hbm, out_hbm,
              idx, ones_v, acc, sem):
    c = lax.axis_index("c")
    s = lax.axis_index("s")
    w = s * NCORE + c
    pltpu.sync_copy(ones_hbm, ones_v)
    for slot, src_hbm in ((0, srcd_hbm), (1, dst_hbm)):
        pltpu.sync_copy(zeros_hbm, acc.at[pl.ds(s * RPW, RPW)])
        plsc.subcore_barrier()
        for half in range(2):
            pltpu.sync_copy(src_hbm.at[w, half], idx)

            def body(g, carry):
                for q in range(_DEG_K):
                    pltpu.async_copy(ones_v, acc.at[idx.at[g * _DEG_K + q]],
                                     sem, add=True)
                for q in range(_DEG_K):
                    pltpu.make_async_copy(ones_hbm, ones_v, sem).wait()
                return carry

            lax.fori_loop(0, HALF // _DEG_K, body, 0)
        plsc.subcore_barrier()
        pltpu.sync_copy(acc.at[pl.ds(s * RPW, RPW)],
                        out_hbm.at[c, slot, pl.ds(s * RPW, RPW)])
        plsc.subcore_barrier()


@functools.cache
def _deg_call():
    return pl.kernel(
        _deg_body,
        out_type=jax.ShapeDtypeStruct((NCORE, 2, NPAD, 16), jnp.float32),
        mesh=_mesh(),
        scratch_types=[
            pltpu.VMEM((SLAB, CHUNK), jnp.int32),
            pltpu.VMEM((CHUNK, 16), jnp.float32),
            pltpu.VMEM_SHARED((NPAD, 16), jnp.float32),
            pltpu.SemaphoreType.DMA,
        ],
        compiler_params=pltpu.CompilerParams(use_tc_tiling_on_sc=False),
    )


def _seg_body(srcg_hbm, dst_hbm, table_hbm, zeros_hbm, out_hbm,
              idxs, idxd, rows0, rows1, acc, sem0, sem1):
    c = lax.axis_index("c")
    s = lax.axis_index("s")
    w = s * NCORE + c
    pltpu.sync_copy(zeros_hbm, acc.at[pl.ds(s * RPW, RPW)])
    plsc.subcore_barrier()
    for half in range(2):
        pltpu.sync_copy(srcg_hbm.at[w, half], idxs)
        pltpu.sync_copy(dst_hbm.at[w, half], idxd)
        pltpu.async_copy(table_hbm.at[idxs.at[0]], rows0, sem0)
        pltpu.async_copy(table_hbm.at[idxs.at[1]], rows1, sem1)

        def body(i, carry):
            j0 = 2 * i
            pltpu.make_async_copy(table_hbm.at[idxs.at[j0]], rows0, sem0).wait()
            pltpu.sync_copy(rows0, acc.at[idxd.at[j0]], add=True)
            pltpu.async_copy(table_hbm.at[idxs.at[j0 + 2]], rows0, sem0)
            pltpu.make_async_copy(table_hbm.at[idxs.at[j0 + 1]], rows1, sem1).wait()
            pltpu.sync_copy(rows1, acc.at[idxd.at[j0 + 1]], add=True)
            pltpu.async_copy(table_hbm.at[idxs.at[j0 + 3]], rows1, sem1)
            return carry

        lax.fori_loop(0, HALF // 2, body, 0)
        pltpu.make_async_copy(table_hbm.at[idxs.at[HALF]], rows0, sem0).wait()
        pltpu.make_async_copy(table_hbm.at[idxs.at[HALF + 1]], rows1, sem1).wait()
    plsc.subcore_barrier()
    pltpu.sync_copy(acc.at[pl.ds(s * RPW, RPW)],
                    out_hbm.at[c, pl.ds(s * RPW, RPW)])


@functools.cache
def _seg_call(d):
    return pl.kernel(
        _seg_body,
        out_type=jax.ShapeDtypeStruct((NCORE, NPAD, d), jnp.float32),
        mesh=_mesh(),
        scratch_types=[
            pltpu.VMEM((SLAB, CHUNK), jnp.int32),
            pltpu.VMEM((SLAB, CHUNK), jnp.int32),
            pltpu.VMEM((CHUNK, d), jnp.float32),
            pltpu.VMEM((CHUNK, d), jnp.float32),
            pltpu.VMEM_SHARED((NPAD, d), jnp.float32),
            pltpu.SemaphoreType.DMA,
            pltpu.SemaphoreType.DMA,
        ],
        compiler_params=pltpu.CompilerParams(use_tc_tiling_on_sc=False),
    )


def _norm_from(deg2):
    return lax.rsqrt(jnp.maximum(deg2, 1.0))[:, 0:1]


def _mm1_body(x_ref, deg_ref, w1_ref, out_ref):
    no = _norm_from(deg_ref[0, 0] + deg_ref[1, 0])
    w1 = w1_ref[...]
    parts = [jnp.dot(x_ref[t], w1, preferred_element_type=jnp.float32)
             for t in range(T)]
    out_ref[...] = jnp.concatenate(parts, axis=1) * no


def _mm2_body(m1_ref, deg_ref, b1_ref, w2_ref, out_ref):
    no = _norm_from(deg_ref[0, 0] + deg_ref[1, 0])
    ni = _norm_from(deg_ref[0, 1] + deg_ref[1, 1])
    m1 = m1_ref[0] + m1_ref[1]
    h1 = jnp.maximum(m1 * ni + b1_ref[...], 0.0) * no
    out_ref[...] = jnp.dot(h1, w2_ref[...], preferred_element_type=jnp.float32)


def _lstm_body(m2_ref, deg_ref, b2_ref, wih_ref, whh_ref, bg_ref,
               wout_ref, bo_ref, pool_ref, out_ref):
    ni = _norm_from(deg_ref[0, 1] + deg_ref[1, 1])
    m2 = m2_ref[0] + m2_ref[1]
    h2 = jnp.maximum(m2 * ni + b2_ref[...], 0.0)
    wih = wih_ref[...]
    whh = whh_ref[...]
    bg = bg_ref[...]
    h = jnp.zeros((BLK, 8), jnp.float32)
    c = jnp.zeros((BLK, 8), jnp.float32)
    for t in range(T):
        xt = h2[:, H2 * t:H2 * t + H2]
        g = (jnp.dot(xt, wih, preferred_element_type=jnp.float32)
             + jnp.dot(h, whh, preferred_element_type=jnp.float32) + bg)
        i = jax.nn.sigmoid(g[:, 0:8])
        f = jax.nn.sigmoid(g[:, 8:16])
        gg = jnp.tanh(g[:, 16:24])
        o = jax.nn.sigmoid(g[:, 24:32])
        c = f * c + i * gg
        h = o * jnp.tanh(c)
    bmax = jnp.max(h, axis=0, keepdims=True)

    @pl.when(pl.program_id(0) == 0)
    def _():
        pool_ref[...] = bmax

    @pl.when(pl.program_id(0) > 0)
    def _():
        pool_ref[...] = jnp.maximum(pool_ref[...], bmax)

    @pl.when(pl.program_id(0) == NBLK - 1)
    def _():
        out_ref[...] = jax.nn.sigmoid(
            jnp.dot(pool_ref[...], wout_ref[...],
                    preferred_element_type=jnp.float32) + bo_ref[...])


_DEG_SPEC = pl.BlockSpec((NCORE, 2, BLK, 16), lambda n: (0, 0, n, 0))


_mm1_call = pl.pallas_call(
    _mm1_body,
    grid=(NBLK,),
    in_specs=[
        pl.BlockSpec((T, BLK, F_IN), lambda n: (0, n, 0)),
        _DEG_SPEC,
        pl.BlockSpec((F_IN, H1), lambda n: (0, 0)),
    ],
    out_specs=pl.BlockSpec((BLK, D1), lambda n: (n, 0)),
    out_shape=jax.ShapeDtypeStruct((N, D1), jnp.float32),
)

_mm2_call = pl.pallas_call(
    _mm2_body,
    grid=(NBLK,),
    in_specs=[
        pl.BlockSpec((NCORE, BLK, D1), lambda n: (0, n, 0)),
        _DEG_SPEC,
        pl.BlockSpec((1, D1), lambda n: (0, 0)),
        pl.BlockSpec((D1, D2), lambda n: (0, 0)),
    ],
    out_specs=pl.BlockSpec((BLK, D2), lambda n: (n, 0)),
    out_shape=jax.ShapeDtypeStruct((N, D2), jnp.float32),
)

_lstm_call = pl.pallas_call(
    _lstm_body,
    grid=(NBLK,),
    in_specs=[
        pl.BlockSpec((NCORE, BLK, D2), lambda n: (0, n, 0)),
        _DEG_SPEC,
        pl.BlockSpec((1, D2), lambda n: (0, 0)),
        pl.BlockSpec((H2, 32), lambda n: (0, 0)),
        pl.BlockSpec((8, 32), lambda n: (0, 0)),
        pl.BlockSpec((1, 32), lambda n: (0, 0)),
        pl.BlockSpec((8, 4), lambda n: (0, 0)),
        pl.BlockSpec((1, 4), lambda n: (0, 0)),
    ],
    out_specs=[
        pl.BlockSpec((1, 8), lambda n: (0, 0)),
        pl.BlockSpec((1, 4), lambda n: (0, 0)),
    ],
    out_shape=[
        jax.ShapeDtypeStruct((1, 8), jnp.float32),
        jax.ShapeDtypeStruct((1, 4), jnp.float32),
    ],
)


def _prep_indices(edge_index):
    src = edge_index[0]
    dst = edge_index[1]
    e = src.shape[0]
    pad = NW * EPW - e
    srcg = jnp.concatenate([src, jnp.zeros((pad,), jnp.int32)])
    srcd = jnp.concatenate([src, jnp.full((pad,), N, jnp.int32)])
    dst3 = jnp.concatenate([dst, jnp.full((pad,), N, jnp.int32)])
    srcg = srcg.reshape(NW, 2, HALF, CHUNK)
    srcd = srcd.reshape(NW, 2, HALF, CHUNK)
    dst3 = dst3.reshape(NW, 2, HALF, CHUNK)
    look0 = jnp.zeros((NW, 2, NLOOK, CHUNK), jnp.int32)
    lookn = jnp.full((NW, 2, NLOOK, CHUNK), N, jnp.int32)
    srcg = jnp.concatenate([srcg, look0], axis=2)
    srcd = jnp.concatenate([srcd, lookn], axis=2)
    dst3 = jnp.concatenate([dst3, lookn], axis=2)
    return srcg, srcd, dst3


@jax.jit
def kernel(x, edge_index, W1, b1, W2, b2, W_ih, W_hh, b_ih, b_hh,
           W_out, b_out):
    srcg, srcd, dst3 = _prep_indices(edge_index)

    ones16 = jnp.ones((CHUNK, 16), jnp.float32)
    zeros16 = jnp.zeros((RPW, 16), jnp.float32)
    zeros64 = jnp.zeros((RPW, D1), jnp.float32)
    zeros128 = jnp.zeros((RPW, D2), jnp.float32)

    degp = _deg_call()(srcd, dst3, ones16, zeros16)
    p1s = _mm1_call(x, degp, W1)
    m1p = _seg_call(D1)(srcg, dst3, p1s, zeros64)
    w2big = jnp.kron(jnp.eye(T, dtype=jnp.float32), W2)
    p2s = _mm2_call(m1p, degp, jnp.tile(b1, T)[None], w2big)
    m2p = _seg_call(D2)(srcg, dst3, p2s, zeros128)
    _, out = _lstm_call(m2p, degp, jnp.tile(b2, T)[None],
                        W_ih.T, W_hh.T, (b_ih + b_hh)[None],
                        W_out.T, b_out[None])
    return out


# speedup vs baseline: 1.8091x; 1.8091x over previous
import functools

import jax
import jax.numpy as jnp
from jax import lax
from jax.experimental import pallas as pl
from jax.experimental.pallas import tpu as pltpu
from jax.experimental.pallas import tpu_sc as plsc

N = 10000
T = 8
F_IN = 128
H1 = 8
H2 = 16
D1 = T * H1
D2 = T * H2
NCORE = 2
NSUB = 16
NW = NCORE * NSUB
CHUNK = 128
NCHUNK = 80
EPW = NCHUNK * CHUNK
HALF = NCHUNK // 2
NLOOK = 2
SLAB = HALF + NLOOK
NPAD = 10016
RPW = NPAD // NSUB
BLK = 1000
NBLK = N // BLK

def _mesh():
    return plsc.VectorSubcoreMesh(
        core_axis_name="c", subcore_axis_name="s",
        num_cores=NCORE, num_subcores=NSUB,
    )


_DEG_K = 8


def _deg_body(srcd_hbm, dst_hbm, ones_hbm, zeros_hbm, out_hbm,
              idx, ones_v, acc, sem):
    c = lax.axis_index("c")
    s = lax.axis_index("s")
    w = s * NCORE + c
    pltpu.sync_copy(ones_hbm, ones_v)
    for slot, src_hbm in ((0, srcd_hbm), (1, dst_hbm)):
        pltpu.sync_copy(zeros_hbm, acc.at[pl.ds(s * RPW, RPW)])
        plsc.subcore_barrier()
        for half in range(2):
            pltpu.sync_copy(src_hbm.at[w, half], idx)

            def body(g, carry):
                for q in range(_DEG_K):
                    pltpu.async_copy(ones_v, acc.at[idx.at[g * _DEG_K + q]],
                                     sem, add=True)
                for q in range(_DEG_K):
                    pltpu.make_async_copy(ones_hbm, ones_v, sem).wait()
                return carry

            lax.fori_loop(0, HALF // _DEG_K, body, 0)
        plsc.subcore_barrier()
        pltpu.sync_copy(acc.at[pl.ds(s * RPW, RPW)],
                        out_hbm.at[c, slot, pl.ds(s * RPW, RPW)])
        plsc.subcore_barrier()


@functools.cache
def _deg_call():
    return pl.kernel(
        _deg_body,
        out_type=jax.ShapeDtypeStruct((NCORE, 2, NPAD, 16), jnp.float32),
        mesh=_mesh(),
        scratch_types=[
            pltpu.VMEM((SLAB, CHUNK), jnp.int32),
            pltpu.VMEM((CHUNK, 16), jnp.float32),
            pltpu.VMEM_SHARED((NPAD, 16), jnp.float32),
            pltpu.SemaphoreType.DMA,
        ],
        compiler_params=pltpu.CompilerParams(use_tc_tiling_on_sc=False),
    )


def _seg_body(srcg_hbm, dst_hbm, table_hbm, zeros_hbm, out_hbm,
              idxs, idxd, rows0, rows1, acc, sem0, sem1):
    c = lax.axis_index("c")
    s = lax.axis_index("s")
    w = s * NCORE + c
    pltpu.sync_copy(zeros_hbm, acc.at[pl.ds(s * RPW, RPW)])
    plsc.subcore_barrier()
    for half in range(2):
        pltpu.sync_copy(srcg_hbm.at[w, half], idxs)
        pltpu.sync_copy(dst_hbm.at[w, half], idxd)

        def body(j, carry):
            pltpu.async_copy(table_hbm.at[idxs.at[j]], rows0, sem0).wait()
            pltpu.sync_copy(rows0, acc.at[idxd.at[j]], add=True)
            return carry

        lax.fori_loop(0, HALF, body, 0)
    plsc.subcore_barrier()
    pltpu.sync_copy(acc.at[pl.ds(s * RPW, RPW)],
                    out_hbm.at[c, pl.ds(s * RPW, RPW)])


@functools.cache
def _seg_call(d):
    return pl.kernel(
        _seg_body,
        out_type=jax.ShapeDtypeStruct((NCORE, NPAD, d), jnp.float32),
        mesh=_mesh(),
        scratch_types=[
            pltpu.VMEM((SLAB, CHUNK), jnp.int32),
            pltpu.VMEM((SLAB, CHUNK), jnp.int32),
            pltpu.VMEM((CHUNK, d), jnp.float32),
            pltpu.VMEM((CHUNK, d), jnp.float32),
            pltpu.VMEM_SHARED((NPAD, d), jnp.float32),
            pltpu.SemaphoreType.DMA,
            pltpu.SemaphoreType.DMA,
        ],
        compiler_params=pltpu.CompilerParams(use_tc_tiling_on_sc=False),
    )


def _norm_from(deg2):
    return lax.rsqrt(jnp.maximum(deg2, 1.0))[:, 0:1]


def _mm1_body(x_ref, deg_ref, w1_ref, out_ref):
    no = _norm_from(deg_ref[0, 0] + deg_ref[1, 0])
    w1 = w1_ref[...]
    parts = [jnp.dot(x_ref[t], w1, preferred_element_type=jnp.float32)
             for t in range(T)]
    out_ref[...] = jnp.concatenate(parts, axis=1) * no


def _mm2_body(m1_ref, deg_ref, b1_ref, w2_ref, out_ref):
    no = _norm_from(deg_ref[0, 0] + deg_ref[1, 0])
    ni = _norm_from(deg_ref[0, 1] + deg_ref[1, 1])
    m1 = m1_ref[0] + m1_ref[1]
    h1 = jnp.maximum(m1 * ni + b1_ref[...], 0.0) * no
    out_ref[...] = jnp.dot(h1, w2_ref[...], preferred_element_type=jnp.float32)


def _lstm_body(m2_ref, deg_ref, b2_ref, wih_ref, whh_ref, bg_ref,
               wout_ref, bo_ref, pool_ref, out_ref):
    ni = _norm_from(deg_ref[0, 1] + deg_ref[1, 1])
    m2 = m2_ref[0] + m2_ref[1]
    h2 = jnp.maximum(m2 * ni + b2_ref[...], 0.0)
    wih = wih_ref[...]
    whh = whh_ref[...]
    bg = bg_ref[...]
    h = jnp.zeros((BLK, 8), jnp.float32)
    c = jnp.zeros((BLK, 8), jnp.float32)
    for t in range(T):
        xt = h2[:, H2 * t:H2 * t + H2]
        g = (jnp.dot(xt, wih, preferred_element_type=jnp.float32)
             + jnp.dot(h, whh, preferred_element_type=jnp.float32) + bg)
        i = jax.nn.sigmoid(g[:, 0:8])
        f = jax.nn.sigmoid(g[:, 8:16])
        gg = jnp.tanh(g[:, 16:24])
        o = jax.nn.sigmoid(g[:, 24:32])
        c = f * c + i * gg
        h = o * jnp.tanh(c)
    bmax = jnp.max(h, axis=0, keepdims=True)

    @pl.when(pl.program_id(0) == 0)
    def _():
        pool_ref[...] = bmax

    @pl.when(pl.program_id(0) > 0)
    def _():
        pool_ref[...] = jnp.maximum(pool_ref[...], bmax)

    @pl.when(pl.program_id(0) == NBLK - 1)
    def _():
        out_ref[...] = jax.nn.sigmoid(
            jnp.dot(pool_ref[...], wout_ref[...],
                    preferred_element_type=jnp.float32) + bo_ref[...])


_DEG_SPEC = pl.BlockSpec((NCORE, 2, BLK, 16), lambda n: (0, 0, n, 0))


_mm1_call = pl.pallas_call(
    _mm1_body,
    grid=(NBLK,),
    in_specs=[
        pl.BlockSpec((T, BLK, F_IN), lambda n: (0, n, 0)),
        _DEG_SPEC,
        pl.BlockSpec((F_IN, H1), lambda n: (0, 0)),
    ],
    out_specs=pl.BlockSpec((BLK, D1), lambda n: (n, 0)),
    out_shape=jax.ShapeDtypeStruct((N, D1), jnp.float32),
)

_mm2_call = pl.pallas_call(
    _mm2_body,
    grid=(NBLK,),
    in_specs=[
        pl.BlockSpec((NCORE, BLK, D1), lambda n: (0, n, 0)),
        _DEG_SPEC,
        pl.BlockSpec((1, D1), lambda n: (0, 0)),
        pl.BlockSpec((D1, D2), lambda n: (0, 0)),
    ],
    out_specs=pl.BlockSpec((BLK, D2), lambda n: (n, 0)),
    out_shape=jax.ShapeDtypeStruct((N, D2), jnp.float32),
)

_lstm_call = pl.pallas_call(
    _lstm_body,
    grid=(NBLK,),
    in_specs=[
        pl.BlockSpec((NCORE, BLK, D2), lambda n: (0, n, 0)),
        _DEG_SPEC,
        pl.BlockSpec((1, D2), lambda n: (0, 0)),
        pl.BlockSpec((H2, 32), lambda n: (0, 0)),
        pl.BlockSpec((8, 32), lambda n: (0, 0)),
        pl.BlockSpec((1, 32), lambda n: (0, 0)),
        pl.BlockSpec((8, 4), lambda n: (0, 0)),
        pl.BlockSpec((1, 4), lambda n: (0, 0)),
    ],
    out_specs=[
        pl.BlockSpec((1, 8), lambda n: (0, 0)),
        pl.BlockSpec((1, 4), lambda n: (0, 0)),
    ],
    out_shape=[
        jax.ShapeDtypeStruct((1, 8), jnp.float32),
        jax.ShapeDtypeStruct((1, 4), jnp.float32),
    ],
)


def _prep_indices(edge_index):
    src = edge_index[0]
    dst = edge_index[1]
    e = src.shape[0]
    pad = NW * EPW - e
    srcg = jnp.concatenate([src, jnp.zeros((pad,), jnp.int32)])
    srcd = jnp.concatenate([src, jnp.full((pad,), N, jnp.int32)])
    dst3 = jnp.concatenate([dst, jnp.full((pad,), N, jnp.int32)])
    srcg = srcg.reshape(NW, 2, HALF, CHUNK)
    srcd = srcd.reshape(NW, 2, HALF, CHUNK)
    dst3 = dst3.reshape(NW, 2, HALF, CHUNK)
    look0 = jnp.zeros((NW, 2, NLOOK, CHUNK), jnp.int32)
    lookn = jnp.full((NW, 2, NLOOK, CHUNK), N, jnp.int32)
    srcg = jnp.concatenate([srcg, look0], axis=2)
    srcd = jnp.concatenate([srcd, lookn], axis=2)
    dst3 = jnp.concatenate([dst3, lookn], axis=2)
    return srcg, srcd, dst3


@jax.jit
def kernel(x, edge_index, W1, b1, W2, b2, W_ih, W_hh, b_ih, b_hh,
           W_out, b_out):
    srcg, srcd, dst3 = _prep_indices(edge_index)

    ones16 = jnp.ones((CHUNK, 16), jnp.float32)
    zeros16 = jnp.zeros((RPW, 16), jnp.float32)
    zeros64 = jnp.zeros((RPW, D1), jnp.float32)
    zeros128 = jnp.zeros((RPW, D2), jnp.float32)

    degp = _deg_call()(srcd, dst3, ones16, zeros16)
    p1s = _mm1_call(x, degp, W1)
    m1p = _seg_call(D1)(srcg, dst3, p1s, zeros64)
    w2big = jnp.kron(jnp.eye(T, dtype=jnp.float32), W2)
    p2s = _mm2_call(m1p, degp, jnp.tile(b1, T)[None], w2big)
    m2p = _seg_call(D2)(srcg, dst3, p2s, zeros128)
    _, out = _lstm_call(m2p, degp, jnp.tile(b2, T)[None],
                        W_ih.T, W_hh.T, (b_ih + b_hh)[None],
                        W_out.T, b_out[None])
    return out


# speedup vs baseline: 1.8380x; 1.0159x over previous
import functools

import jax
import jax.numpy as jnp
from jax import lax
from jax.experimental import pallas as pl
from jax.experimental.pallas import tpu as pltpu
from jax.experimental.pallas import tpu_sc as plsc

N = 10000
T = 8
F_IN = 128
H1 = 8
H2 = 16
D1 = T * H1
D2 = T * H2
NCORE = 2
NSUB = 16
NW = NCORE * NSUB
CHUNK = 128
NCHUNK = 80
EPW = NCHUNK * CHUNK
NPAD = 10240
RPW = NPAD // NSUB
BLK = 1000
NBLK = N // BLK

def _mesh():
    return plsc.VectorSubcoreMesh(
        core_axis_name="c", subcore_axis_name="s",
        num_cores=NCORE, num_subcores=NSUB,
    )


_DEG_K = 8


def _deg_body(srcd_hbm, dst_hbm, ones_hbm, zeros_hbm, out_hbm,
              idx, ones_v, acc, sem):
    c = lax.axis_index("c")
    s = lax.axis_index("s")
    w = s * NCORE + c
    pltpu.sync_copy(ones_hbm, ones_v)
    for slot, src_hbm in ((0, srcd_hbm), (1, dst_hbm)):
        pltpu.sync_copy(zeros_hbm, acc.at[pl.ds(s * RPW, RPW)])
        pltpu.sync_copy(src_hbm.at[w], idx)
        plsc.subcore_barrier()

        def body(g, carry):
            for q in range(_DEG_K):
                pltpu.async_copy(ones_v, acc.at[idx.at[g * _DEG_K + q]],
                                 sem, add=True)
            for q in range(_DEG_K):
                pltpu.make_async_copy(ones_hbm, ones_v, sem).wait()
            return carry

        lax.fori_loop(0, NCHUNK // _DEG_K, body, 0)
        plsc.subcore_barrier()
        pltpu.sync_copy(acc.at[pl.ds(s * RPW, RPW)],
                        out_hbm.at[c, slot, pl.ds(s * RPW, RPW)])
        plsc.subcore_barrier()


@functools.cache
def _deg_call():
    return pl.kernel(
        _deg_body,
        out_type=jax.ShapeDtypeStruct((NCORE, 2, NPAD, 16), jnp.float32),
        mesh=_mesh(),
        scratch_types=[
            pltpu.VMEM((NCHUNK, CHUNK), jnp.int32),
            pltpu.VMEM((CHUNK, 16), jnp.float32),
            pltpu.VMEM_SHARED((NPAD, 16), jnp.float32),
            pltpu.SemaphoreType.DMA,
        ],
        compiler_params=pltpu.CompilerParams(use_tc_tiling_on_sc=False),
    )


def _seg_body(srcg_hbm, dst_hbm, table_hbm, zeros_hbm, out_hbm,
              idxs, idxd, rows0, acc, sem0):
    c = lax.axis_index("c")
    s = lax.axis_index("s")
    w = s * NCORE + c
    pltpu.sync_copy(srcg_hbm.at[w], idxs)
    pltpu.sync_copy(dst_hbm.at[w], idxd)
    pltpu.sync_copy(zeros_hbm, acc.at[pl.ds(s * RPW, RPW)])
    plsc.subcore_barrier()

    def body(j, carry):
        pltpu.async_copy(table_hbm.at[idxs.at[j]], rows0, sem0).wait()
        pltpu.sync_copy(rows0, acc.at[idxd.at[j]], add=True)
        return carry

    lax.fori_loop(0, NCHUNK, body, 0)
    plsc.subcore_barrier()
    pltpu.sync_copy(acc.at[pl.ds(s * RPW, RPW)],
                    out_hbm.at[c, pl.ds(s * RPW, RPW)])


@functools.cache
def _seg_call(d):
    return pl.kernel(
        _seg_body,
        out_type=jax.ShapeDtypeStruct((NCORE, NPAD, d), jnp.float32),
        mesh=_mesh(),
        scratch_types=[
            pltpu.VMEM((NCHUNK, CHUNK), jnp.int32),
            pltpu.VMEM((NCHUNK, CHUNK), jnp.int32),
            pltpu.VMEM((CHUNK, d), jnp.float32),
            pltpu.VMEM_SHARED((NPAD, d), jnp.float32),
            pltpu.SemaphoreType.DMA,
        ],
        compiler_params=pltpu.CompilerParams(use_tc_tiling_on_sc=False),
    )


def _norm_from(deg2):
    return lax.rsqrt(jnp.maximum(deg2, 1.0))[:, 0:1]


def _mm1_body(x_ref, deg_ref, w1_ref, out_ref):
    no = _norm_from(deg_ref[0, 0] + deg_ref[1, 0])
    w1 = w1_ref[...]
    parts = [jnp.dot(x_ref[t], w1, preferred_element_type=jnp.float32)
             for t in range(T)]
    out_ref[...] = jnp.concatenate(parts, axis=1) * no


def _mm2_body(m1_ref, deg_ref, b1_ref, w2_ref, out_ref):
    no = _norm_from(deg_ref[0, 0] + deg_ref[1, 0])
    ni = _norm_from(deg_ref[0, 1] + deg_ref[1, 1])
    m1 = m1_ref[0] + m1_ref[1]
    h1 = jnp.maximum(m1 * ni + b1_ref[...], 0.0) * no
    out_ref[...] = jnp.dot(h1, w2_ref[...], preferred_element_type=jnp.float32)


def _lstm_body(m2_ref, deg_ref, b2_ref, wih_ref, whh_ref, bg_ref,
               wout_ref, bo_ref, pool_ref, out_ref):
    ni = _norm_from(deg_ref[0, 1] + deg_ref[1, 1])
    m2 = m2_ref[0] + m2_ref[1]
    h2 = jnp.maximum(m2 * ni + b2_ref[...], 0.0)
    wih = wih_ref[...]
    whh = whh_ref[...]
    bg = bg_ref[...]
    h = jnp.zeros((BLK, 8), jnp.float32)
    c = jnp.zeros((BLK, 8), jnp.float32)
    for t in range(T):
        xt = h2[:, H2 * t:H2 * t + H2]
        g = (jnp.dot(xt, wih, preferred_element_type=jnp.float32)
             + jnp.dot(h, whh, preferred_element_type=jnp.float32) + bg)
        i = jax.nn.sigmoid(g[:, 0:8])
        f = jax.nn.sigmoid(g[:, 8:16])
        gg = jnp.tanh(g[:, 16:24])
        o = jax.nn.sigmoid(g[:, 24:32])
        c = f * c + i * gg
        h = o * jnp.tanh(c)
    bmax = jnp.max(h, axis=0, keepdims=True)

    @pl.when(pl.program_id(0) == 0)
    def _():
        pool_ref[...] = bmax

    @pl.when(pl.program_id(0) > 0)
    def _():
        pool_ref[...] = jnp.maximum(pool_ref[...], bmax)

    @pl.when(pl.program_id(0) == NBLK - 1)
    def _():
        out_ref[...] = jax.nn.sigmoid(
            jnp.dot(pool_ref[...], wout_ref[...],
                    preferred_element_type=jnp.float32) + bo_ref[...])


_DEG_SPEC = pl.BlockSpec((NCORE, 2, BLK, 16), lambda n: (0, 0, n, 0))


_mm1_call = pl.pallas_call(
    _mm1_body,
    grid=(NBLK,),
    in_specs=[
        pl.BlockSpec((T, BLK, F_IN), lambda n: (0, n, 0)),
        _DEG_SPEC,
        pl.BlockSpec((F_IN, H1), lambda n: (0, 0)),
    ],
    out_specs=pl.BlockSpec((BLK, D1), lambda n: (n, 0)),
    out_shape=jax.ShapeDtypeStruct((N, D1), jnp.float32),
)

_mm2_call = pl.pallas_call(
    _mm2_body,
    grid=(NBLK,),
    in_specs=[
        pl.BlockSpec((NCORE, BLK, D1), lambda n: (0, n, 0)),
        _DEG_SPEC,
        pl.BlockSpec((1, D1), lambda n: (0, 0)),
        pl.BlockSpec((D1, D2), lambda n: (0, 0)),
    ],
    out_specs=pl.BlockSpec((BLK, D2), lambda n: (n, 0)),
    out_shape=jax.ShapeDtypeStruct((N, D2), jnp.float32),
)

_lstm_call = pl.pallas_call(
    _lstm_body,
    grid=(NBLK,),
    in_specs=[
        pl.BlockSpec((NCORE, BLK, D2), lambda n: (0, n, 0)),
        _DEG_SPEC,
        pl.BlockSpec((1, D2), lambda n: (0, 0)),
        pl.BlockSpec((H2, 32), lambda n: (0, 0)),
        pl.BlockSpec((8, 32), lambda n: (0, 0)),
        pl.BlockSpec((1, 32), lambda n: (0, 0)),
        pl.BlockSpec((8, 4), lambda n: (0, 0)),
        pl.BlockSpec((1, 4), lambda n: (0, 0)),
    ],
    out_specs=[
        pl.BlockSpec((1, 8), lambda n: (0, 0)),
        pl.BlockSpec((1, 4), lambda n: (0, 0)),
    ],
    out_shape=[
        jax.ShapeDtypeStruct((1, 8), jnp.float32),
        jax.ShapeDtypeStruct((1, 4), jnp.float32),
    ],
)


def _prep_indices(edge_index):
    src = edge_index[0]
    dst = edge_index[1]
    e = src.shape[0]
    pad = NW * EPW - e
    srcg = jnp.concatenate([src, jnp.zeros((pad,), jnp.int32)])
    srcd = jnp.concatenate([src, jnp.full((pad,), N, jnp.int32)])
    dst3 = jnp.concatenate([dst, jnp.full((pad,), N, jnp.int32)])
    srcg = srcg.reshape(NW, NCHUNK, CHUNK)
    srcd = srcd.reshape(NW, NCHUNK, CHUNK)
    dst3 = dst3.reshape(NW, NCHUNK, CHUNK)
    return srcg, srcd, dst3


@jax.jit
def kernel(x, edge_index, W1, b1, W2, b2, W_ih, W_hh, b_ih, b_hh,
           W_out, b_out):
    srcg, srcd, dst3 = _prep_indices(edge_index)

    ones16 = jnp.ones((CHUNK, 16), jnp.float32)
    zeros16 = jnp.zeros((RPW, 16), jnp.float32)
    zeros64 = jnp.zeros((RPW, D1), jnp.float32)
    zeros128 = jnp.zeros((RPW, D2), jnp.float32)

    degp = _deg_call()(srcd, dst3, ones16, zeros16)
    p1s = _mm1_call(x, degp, W1)
    m1p = _seg_call(D1)(srcg, dst3, p1s, zeros64)
    w2big = jnp.kron(jnp.eye(T, dtype=jnp.float32), W2)
    p2s = _mm2_call(m1p, degp, jnp.tile(b1, T)[None], w2big)
    m2p = _seg_call(D2)(srcg, dst3, p2s, zeros128)
    _, out = _lstm_call(m2p, degp, jnp.tile(b2, T)[None],
                        W_ih.T, W_hh.T, (b_ih + b_hh)[None],
                        W_out.T, b_out[None])
    return out


# speedup vs baseline: 1.8424x; 1.0024x over previous
import functools

import jax
import jax.numpy as jnp
from jax import lax
from jax.experimental import pallas as pl
from jax.experimental.pallas import tpu as pltpu
from jax.experimental.pallas import tpu_sc as plsc

N = 10000
T = 8
F_IN = 128
H1 = 8
H2 = 16
D1 = T * H1
D2 = T * H2
NCORE = 2
NSUB = 16
NW = NCORE * NSUB
CHUNK = 128
NCHUNK = 80
EPW = NCHUNK * CHUNK
NPAD = 10240
RPW = NPAD // NSUB
BLK = 1000
NBLK = N // BLK

def _mesh():
    return plsc.VectorSubcoreMesh(
        core_axis_name="c", subcore_axis_name="s",
        num_cores=NCORE, num_subcores=NSUB,
    )


def _deg_body(srcd_hbm, dst_hbm, ones_hbm, zeros_hbm, out_hbm,
              idxs, idxd, ones_v, acc_a, acc_b):
    c = lax.axis_index("c")
    s = lax.axis_index("s")
    w = s * NCORE + c
    pltpu.sync_copy(srcd_hbm.at[w], idxs)
    pltpu.sync_copy(dst_hbm.at[w], idxd)
    pltpu.sync_copy(ones_hbm, ones_v)
    pltpu.sync_copy(zeros_hbm, acc_a.at[pl.ds(s * RPW, RPW)])
    pltpu.sync_copy(zeros_hbm, acc_b.at[pl.ds(s * RPW, RPW)])
    plsc.subcore_barrier()

    def body(j, carry):
        pltpu.sync_copy(ones_v, acc_a.at[idxs.at[j]], add=True)
        pltpu.sync_copy(ones_v, acc_b.at[idxd.at[j]], add=True)
        return carry

    lax.fori_loop(0, NCHUNK, body, 0)
    plsc.subcore_barrier()
    pltpu.sync_copy(acc_a.at[pl.ds(s * RPW, RPW)],
                    out_hbm.at[c, 0, pl.ds(s * RPW, RPW)])
    pltpu.sync_copy(acc_b.at[pl.ds(s * RPW, RPW)],
                    out_hbm.at[c, 1, pl.ds(s * RPW, RPW)])


@functools.cache
def _deg_call():
    return pl.kernel(
        _deg_body,
        out_type=jax.ShapeDtypeStruct((NCORE, 2, NPAD, 16), jnp.float32),
        mesh=_mesh(),
        scratch_types=[
            pltpu.VMEM((NCHUNK, CHUNK), jnp.int32),
            pltpu.VMEM((NCHUNK, CHUNK), jnp.int32),
            pltpu.VMEM((CHUNK, 16), jnp.float32),
            pltpu.VMEM_SHARED((NPAD, 16), jnp.float32),
            pltpu.VMEM_SHARED((NPAD, 16), jnp.float32),
        ],
        compiler_params=pltpu.CompilerParams(use_tc_tiling_on_sc=False),
    )


def _seg_body(srcg_hbm, dst_hbm, table_hbm, zeros_hbm, out_hbm,
              idxs, idxd, rows0, acc, sem0):
    c = lax.axis_index("c")
    s = lax.axis_index("s")
    w = s * NCORE + c
    pltpu.sync_copy(srcg_hbm.at[w], idxs)
    pltpu.sync_copy(dst_hbm.at[w], idxd)
    pltpu.sync_copy(zeros_hbm, acc.at[pl.ds(s * RPW, RPW)])
    plsc.subcore_barrier()

    def body(j, carry):
        pltpu.async_copy(table_hbm.at[idxs.at[j]], rows0, sem0).wait()
        pltpu.sync_copy(rows0, acc.at[idxd.at[j]], add=True)
        return carry

    lax.fori_loop(0, NCHUNK, body, 0)
    plsc.subcore_barrier()
    pltpu.sync_copy(acc.at[pl.ds(s * RPW, RPW)],
                    out_hbm.at[c, pl.ds(s * RPW, RPW)])


@functools.cache
def _seg_call(d):
    return pl.kernel(
        _seg_body,
        out_type=jax.ShapeDtypeStruct((NCORE, NPAD, d), jnp.float32),
        mesh=_mesh(),
        scratch_types=[
            pltpu.VMEM((NCHUNK, CHUNK), jnp.int32),
            pltpu.VMEM((NCHUNK, CHUNK), jnp.int32),
            pltpu.VMEM((CHUNK, d), jnp.float32),
            pltpu.VMEM_SHARED((NPAD, d), jnp.float32),
            pltpu.SemaphoreType.DMA,
        ],
        compiler_params=pltpu.CompilerParams(use_tc_tiling_on_sc=False),
    )


def _norm_from(deg2):
    return lax.rsqrt(jnp.maximum(deg2, 1.0))[:, 0:1]


def _mm1_body(x_ref, deg_ref, w1_ref, out_ref):
    no = _norm_from(deg_ref[0, 0] + deg_ref[1, 0])
    w1 = w1_ref[...]
    parts = [jnp.dot(x_ref[t], w1, preferred_element_type=jnp.float32)
             for t in range(T)]
    out_ref[...] = jnp.concatenate(parts, axis=1) * no


def _mm2_body(m1_ref, deg_ref, b1_ref, w2_ref, out_ref):
    no = _norm_from(deg_ref[0, 0] + deg_ref[1, 0])
    ni = _norm_from(deg_ref[0, 1] + deg_ref[1, 1])
    m1 = m1_ref[0] + m1_ref[1]
    h1 = jnp.maximum(m1 * ni + b1_ref[...], 0.0) * no
    out_ref[...] = jnp.dot(h1, w2_ref[...], preferred_element_type=jnp.float32)


def _lstm_body(m2_ref, deg_ref, b2_ref, wih_ref, whh_ref, bg_ref,
               wout_ref, bo_ref, pool_ref, out_ref):
    ni = _norm_from(deg_ref[0, 1] + deg_ref[1, 1])
    m2 = m2_ref[0] + m2_ref[1]
    h2 = jnp.maximum(m2 * ni + b2_ref[...], 0.0)
    wih = wih_ref[...]
    whh = whh_ref[...]
    bg = bg_ref[...]
    h = jnp.zeros((BLK, 8), jnp.float32)
    c = jnp.zeros((BLK, 8), jnp.float32)
    for t in range(T):
        xt = h2[:, H2 * t:H2 * t + H2]
        g = (jnp.dot(xt, wih, preferred_element_type=jnp.float32)
             + jnp.dot(h, whh, preferred_element_type=jnp.float32) + bg)
        i = jax.nn.sigmoid(g[:, 0:8])
        f = jax.nn.sigmoid(g[:, 8:16])
        gg = jnp.tanh(g[:, 16:24])
        o = jax.nn.sigmoid(g[:, 24:32])
        c = f * c + i * gg
        h = o * jnp.tanh(c)
    bmax = jnp.max(h, axis=0, keepdims=True)

    @pl.when(pl.program_id(0) == 0)
    def _():
        pool_ref[...] = bmax

    @pl.when(pl.program_id(0) > 0)
    def _():
        pool_ref[...] = jnp.maximum(pool_ref[...], bmax)

    @pl.when(pl.program_id(0) == NBLK - 1)
    def _():
        out_ref[...] = jax.nn.sigmoid(
            jnp.dot(pool_ref[...], wout_ref[...],
                    preferred_element_type=jnp.float32) + bo_ref[...])


_DEG_SPEC = pl.BlockSpec((NCORE, 2, BLK, 16), lambda n: (0, 0, n, 0))


_mm1_call = pl.pallas_call(
    _mm1_body,
    grid=(NBLK,),
    in_specs=[
        pl.BlockSpec((T, BLK, F_IN), lambda n: (0, n, 0)),
        _DEG_SPEC,
        pl.BlockSpec((F_IN, H1), lambda n: (0, 0)),
    ],
    out_specs=pl.BlockSpec((BLK, D1), lambda n: (n, 0)),
    out_shape=jax.ShapeDtypeStruct((N, D1), jnp.float32),
)

_mm2_call = pl.pallas_call(
    _mm2_body,
    grid=(NBLK,),
    in_specs=[
        pl.BlockSpec((NCORE, BLK, D1), lambda n: (0, n, 0)),
        _DEG_SPEC,
        pl.BlockSpec((1, D1), lambda n: (0, 0)),
        pl.BlockSpec((D1, D2), lambda n: (0, 0)),
    ],
    out_specs=pl.BlockSpec((BLK, D2), lambda n: (n, 0)),
    out_shape=jax.ShapeDtypeStruct((N, D2), jnp.float32),
)

_lstm_call = pl.pallas_call(
    _lstm_body,
    grid=(NBLK,),
    in_specs=[
        pl.BlockSpec((NCORE, BLK, D2), lambda n: (0, n, 0)),
        _DEG_SPEC,
        pl.BlockSpec((1, D2), lambda n: (0, 0)),
        pl.BlockSpec((H2, 32), lambda n: (0, 0)),
        pl.BlockSpec((8, 32), lambda n: (0, 0)),
        pl.BlockSpec((1, 32), lambda n: (0, 0)),
        pl.BlockSpec((8, 4), lambda n: (0, 0)),
        pl.BlockSpec((1, 4), lambda n: (0, 0)),
    ],
    out_specs=[
        pl.BlockSpec((1, 8), lambda n: (0, 0)),
        pl.BlockSpec((1, 4), lambda n: (0, 0)),
    ],
    out_shape=[
        jax.ShapeDtypeStruct((1, 8), jnp.float32),
        jax.ShapeDtypeStruct((1, 4), jnp.float32),
    ],
)


def _prep_indices(edge_index):
    src = edge_index[0]
    dst = edge_index[1]
    e = src.shape[0]
    pad = NW * EPW - e
    srcg = jnp.concatenate([src, jnp.zeros((pad,), jnp.int32)])
    srcd = jnp.concatenate([src, jnp.full((pad,), N, jnp.int32)])
    dst3 = jnp.concatenate([dst, jnp.full((pad,), N, jnp.int32)])
    srcg = srcg.reshape(NW, NCHUNK, CHUNK)
    srcd = srcd.reshape(NW, NCHUNK, CHUNK)
    dst3 = dst3.reshape(NW, NCHUNK, CHUNK)
    return srcg, srcd, dst3


@jax.jit
def kernel(x, edge_index, W1, b1, W2, b2, W_ih, W_hh, b_ih, b_hh,
           W_out, b_out):
    srcg, srcd, dst3 = _prep_indices(edge_index)

    ones16 = jnp.ones((CHUNK, 16), jnp.float32)
    zeros16 = jnp.zeros((RPW, 16), jnp.float32)
    zeros64 = jnp.zeros((RPW, D1), jnp.float32)
    zeros128 = jnp.zeros((RPW, D2), jnp.float32)

    degp = _deg_call()(srcd, dst3, ones16, zeros16)
    p1s = _mm1_call(x, degp, W1)
    m1p = _seg_call(D1)(srcg, dst3, p1s, zeros64)
    w2big = jnp.kron(jnp.eye(T, dtype=jnp.float32), W2)
    p2s = _mm2_call(m1p, degp, jnp.tile(b1, T)[None], w2big)
    m2p = _seg_call(D2)(srcg, dst3, p2s, zeros128)
    _, out = _lstm_call(m2p, degp, jnp.tile(b2, T)[None],
                        W_ih.T, W_hh.T, (b_ih + b_hh)[None],
                        W_out.T, b_out[None])
    return out


# speedup vs baseline: 3.6911x; 2.0034x over previous
import functools

import jax
import jax.numpy as jnp
from jax import lax
from jax.experimental import pallas as pl
from jax.experimental.pallas import tpu as pltpu
from jax.experimental.pallas import tpu_sc as plsc

N = 10000
T = 8
F_IN = 128
H1 = 8
H2 = 16
D1 = T * H1
D2 = T * H2
NCORE = 2
NSUB = 16
NW = NCORE * NSUB
CHUNK = 128
NCHUNK = 80
EPW = NCHUNK * CHUNK
NPAD = 10240
RPW = NPAD // NSUB
BLK = 1000
NBLK = N // BLK

def _mesh():
    return plsc.VectorSubcoreMesh(
        core_axis_name="c", subcore_axis_name="s",
        num_cores=NCORE, num_subcores=NSUB,
    )


def _deg_body(srcd_hbm, dst_hbm, ones_hbm, zeros_hbm, out_hbm,
              idxs, idxd, ones_v, acc_a, acc_b):
    c = lax.axis_index("c")
    s = lax.axis_index("s")
    w = s * NCORE + c
    pltpu.sync_copy(srcd_hbm.at[w], idxs)
    pltpu.sync_copy(dst_hbm.at[w], idxd)
    pltpu.sync_copy(ones_hbm, ones_v)
    pltpu.sync_copy(zeros_hbm, acc_a.at[pl.ds(s * RPW, RPW)])
    pltpu.sync_copy(zeros_hbm, acc_b.at[pl.ds(s * RPW, RPW)])
    plsc.subcore_barrier()

    def body(j, carry):
        pltpu.sync_copy(ones_v, acc_a.at[idxs.at[j]], add=True)
        pltpu.sync_copy(ones_v, acc_b.at[idxd.at[j]], add=True)
        return carry

    lax.fori_loop(0, NCHUNK, body, 0)
    plsc.subcore_barrier()
    pltpu.sync_copy(acc_a.at[pl.ds(s * RPW, RPW)],
                    out_hbm.at[c, 0, pl.ds(s * RPW, RPW)])
    pltpu.sync_copy(acc_b.at[pl.ds(s * RPW, RPW)],
                    out_hbm.at[c, 1, pl.ds(s * RPW, RPW)])


@functools.cache
def _deg_call():
    return pl.kernel(
        _deg_body,
        out_type=jax.ShapeDtypeStruct((NCORE, 2, NPAD, 16), jnp.float32),
        mesh=_mesh(),
        scratch_types=[
            pltpu.VMEM((NCHUNK, CHUNK), jnp.int32),
            pltpu.VMEM((NCHUNK, CHUNK), jnp.int32),
            pltpu.VMEM((CHUNK, 16), jnp.float32),
            pltpu.VMEM_SHARED((NPAD, 16), jnp.float32),
            pltpu.VMEM_SHARED((NPAD, 16), jnp.float32),
        ],
        compiler_params=pltpu.CompilerParams(use_tc_tiling_on_sc=False),
    )


def _seg_body(srcg_hbm, dst_hbm, table_hbm, zeros_hbm, out_hbm,
              idxs, idxd, rows0, acc, sem0):
    c = lax.axis_index("c")
    s = lax.axis_index("s")
    w = s * NCORE + c
    pltpu.sync_copy(srcg_hbm.at[w], idxs)
    pltpu.sync_copy(dst_hbm.at[w], idxd)
    pltpu.sync_copy(zeros_hbm, acc.at[pl.ds(s * RPW, RPW)])
    plsc.subcore_barrier()

    def body(j, carry):
        pltpu.async_copy(table_hbm.at[idxs.at[j]], rows0, sem0).wait()
        pltpu.sync_copy(rows0, acc.at[idxd.at[j]], add=True)
        return carry

    lax.fori_loop(0, NCHUNK, body, 0)
    plsc.subcore_barrier()
    pltpu.sync_copy(acc.at[pl.ds(s * RPW, RPW)],
                    out_hbm.at[c, pl.ds(s * RPW, RPW)])


@functools.cache
def _seg_call(d):
    return pl.kernel(
        _seg_body,
        out_type=jax.ShapeDtypeStruct((NCORE, NPAD, d), jnp.float32),
        mesh=_mesh(),
        scratch_types=[
            pltpu.VMEM((NCHUNK, CHUNK), jnp.int32),
            pltpu.VMEM((NCHUNK, CHUNK), jnp.int32),
            pltpu.VMEM((CHUNK, d), jnp.float32),
            pltpu.VMEM_SHARED((NPAD, d), jnp.float32),
            pltpu.SemaphoreType.DMA,
        ],
        compiler_params=pltpu.CompilerParams(use_tc_tiling_on_sc=False),
    )


def _norm_from(deg2):
    return lax.rsqrt(jnp.maximum(deg2, 1.0))[:, 0:1]


def _mm1_body(x_ref, deg_ref, w1_ref, out_ref):
    no = _norm_from(deg_ref[0, 0] + deg_ref[1, 0])
    w1 = w1_ref[...]
    parts = [jnp.dot(x_ref[t], w1, preferred_element_type=jnp.float32)
             for t in range(T)]
    out_ref[...] = jnp.concatenate(parts, axis=1) * no


def _mm2_body(m1_ref, deg_ref, b1_ref, w2_ref, out_ref):
    no = _norm_from(deg_ref[0, 0] + deg_ref[1, 0])
    ni = _norm_from(deg_ref[0, 1] + deg_ref[1, 1])
    m1 = m1_ref[0] + m1_ref[1]
    h1 = jnp.maximum(m1 * ni + b1_ref[...], 0.0) * no
    out_ref[...] = jnp.dot(h1, w2_ref[...], preferred_element_type=jnp.float32)


def _lstm_body(m2_ref, deg_ref, b2_ref, wih_ref, whh_ref, bg_ref,
               wout_ref, bo_ref, pool_ref, out_ref):
    ni = _norm_from(deg_ref[0, 1] + deg_ref[1, 1])
    m2 = m2_ref[0] + m2_ref[1]
    h2 = jnp.maximum(m2 * ni + b2_ref[...], 0.0)
    wih = wih_ref[...]
    whh = whh_ref[...]
    bg = bg_ref[...]
    h = jnp.zeros((BLK, 8), jnp.float32)
    c = jnp.zeros((BLK, 8), jnp.float32)
    for t in range(T):
        xt = h2[:, H2 * t:H2 * t + H2]
        g = (jnp.dot(xt, wih, preferred_element_type=jnp.float32)
             + jnp.dot(h, whh, preferred_element_type=jnp.float32) + bg)
        i = jax.nn.sigmoid(g[:, 0:8])
        f = jax.nn.sigmoid(g[:, 8:16])
        gg = jnp.tanh(g[:, 16:24])
        o = jax.nn.sigmoid(g[:, 24:32])
        c = f * c + i * gg
        h = o * jnp.tanh(c)
    bmax = jnp.max(h, axis=0, keepdims=True)

    @pl.when(pl.program_id(0) == 0)
    def _():
        pool_ref[...] = bmax

    @pl.when(pl.program_id(0) > 0)
    def _():
        pool_ref[...] = jnp.maximum(pool_ref[...], bmax)

    @pl.when(pl.program_id(0) == NBLK - 1)
    def _():
        out_ref[...] = jax.nn.sigmoid(
            jnp.dot(pool_ref[...], wout_ref[...],
                    preferred_element_type=jnp.float32) + bo_ref[...])


_DEG_SPEC = pl.BlockSpec((NCORE, 2, BLK, 16), lambda n: (0, 0, n, 0))


_mm1_call = pl.pallas_call(
    _mm1_body,
    grid=(NBLK,),
    in_specs=[
        pl.BlockSpec((T, BLK, F_IN), lambda n: (0, n, 0)),
        _DEG_SPEC,
        pl.BlockSpec((F_IN, H1), lambda n: (0, 0)),
    ],
    out_specs=pl.BlockSpec((BLK, D1), lambda n: (n, 0)),
    out_shape=jax.ShapeDtypeStruct((N, D1), jnp.float32),
)

_mm2_call = pl.pallas_call(
    _mm2_body,
    grid=(NBLK,),
    in_specs=[
        pl.BlockSpec((NCORE, BLK, D1), lambda n: (0, n, 0)),
        _DEG_SPEC,
        pl.BlockSpec((1, D1), lambda n: (0, 0)),
        pl.BlockSpec((D1, D2), lambda n: (0, 0)),
    ],
    out_specs=pl.BlockSpec((BLK, D2), lambda n: (n, 0)),
    out_shape=jax.ShapeDtypeStruct((N, D2), jnp.float32),
)

_lstm_call = pl.pallas_call(
    _lstm_body,
    grid=(NBLK,),
    in_specs=[
        pl.BlockSpec((NCORE, BLK, D2), lambda n: (0, n, 0)),
        _DEG_SPEC,
        pl.BlockSpec((1, D2), lambda n: (0, 0)),
        pl.BlockSpec((H2, 32), lambda n: (0, 0)),
        pl.BlockSpec((8, 32), lambda n: (0, 0)),
        pl.BlockSpec((1, 32), lambda n: (0, 0)),
        pl.BlockSpec((8, 4), lambda n: (0, 0)),
        pl.BlockSpec((1, 4), lambda n: (0, 0)),
    ],
    out_specs=[
        pl.BlockSpec((1, 8), lambda n: (0, 0)),
        pl.BlockSpec((1, 4), lambda n: (0, 0)),
    ],
    out_shape=[
        jax.ShapeDtypeStruct((1, 8), jnp.float32),
        jax.ShapeDtypeStruct((1, 4), jnp.float32),
    ],
)


def _prep_indices(edge_index):
    src = edge_index[0]
    dst = edge_index[1]
    e = src.shape[0]
    pad = NW * EPW - e
    k = jnp.arange(pad, dtype=jnp.int32)
    trash = N + k % (NPAD - N)
    srcg = jnp.concatenate([src, k % 4096])
    srcd = jnp.concatenate([src, trash])
    dst3 = jnp.concatenate([dst, trash])
    srcg = srcg.reshape(NW, NCHUNK, CHUNK)
    srcd = srcd.reshape(NW, NCHUNK, CHUNK)
    dst3 = dst3.reshape(NW, NCHUNK, CHUNK)
    return srcg, srcd, dst3


@jax.jit
def kernel(x, edge_index, W1, b1, W2, b2, W_ih, W_hh, b_ih, b_hh,
           W_out, b_out):
    srcg, srcd, dst3 = _prep_indices(edge_index)

    ones16 = jnp.ones((CHUNK, 16), jnp.float32)
    zeros16 = jnp.zeros((RPW, 16), jnp.float32)
    zeros64 = jnp.zeros((RPW, D1), jnp.float32)
    zeros128 = jnp.zeros((RPW, D2), jnp.float32)

    degp = _deg_call()(srcd, dst3, ones16, zeros16)
    p1s = _mm1_call(x, degp, W1)
    m1p = _seg_call(D1)(srcg, dst3, p1s, zeros64)
    w2big = jnp.kron(jnp.eye(T, dtype=jnp.float32), W2)
    p2s = _mm2_call(m1p, degp, jnp.tile(b1, T)[None], w2big)
    m2p = _seg_call(D2)(srcg, dst3, p2s, zeros128)
    _, out = _lstm_call(m2p, degp, jnp.tile(b2, T)[None],
                        W_ih.T, W_hh.T, (b_ih + b_hh)[None],
                        W_out.T, b_out[None])
    return out


# speedup vs baseline: 3.7549x; 1.0173x over previous
import functools

import jax
import jax.numpy as jnp
from jax import lax
from jax.experimental import pallas as pl
from jax.experimental.pallas import tpu as pltpu
from jax.experimental.pallas import tpu_sc as plsc

N = 10000
T = 8
F_IN = 128
H1 = 8
H2 = 16
D1 = T * H1
D2 = T * H2
NCORE = 2
NSUB = 16
NW = NCORE * NSUB
CHUNK = 128
NCHUNK = 80
EPW = NCHUNK * CHUNK
NPAD = 10240
RPW = NPAD // NSUB
BLK = 1000
NBLK = N // BLK

def _mesh():
    return plsc.VectorSubcoreMesh(
        core_axis_name="c", subcore_axis_name="s",
        num_cores=NCORE, num_subcores=NSUB,
    )


def _deg_body(srcd_hbm, dst_hbm, ones_hbm, zeros_hbm, out_hbm,
              idxs, idxd, ones_v, acc_a, acc_b):
    c = lax.axis_index("c")
    s = lax.axis_index("s")
    w = s * NCORE + c
    pltpu.sync_copy(srcd_hbm.at[w], idxs)
    pltpu.sync_copy(dst_hbm.at[w], idxd)
    pltpu.sync_copy(ones_hbm, ones_v)
    pltpu.sync_copy(zeros_hbm, acc_a.at[pl.ds(s * RPW, RPW)])
    pltpu.sync_copy(zeros_hbm, acc_b.at[pl.ds(s * RPW, RPW)])
    plsc.subcore_barrier()

    def body(j, carry):
        pltpu.sync_copy(ones_v, acc_a.at[idxs.at[j]], add=True)
        pltpu.sync_copy(ones_v, acc_b.at[idxd.at[j]], add=True)
        return carry

    lax.fori_loop(0, NCHUNK, body, 0)
    plsc.subcore_barrier()
    pltpu.sync_copy(acc_a.at[pl.ds(s * RPW, RPW)],
                    out_hbm.at[c, 0, pl.ds(s * RPW, RPW)])
    pltpu.sync_copy(acc_b.at[pl.ds(s * RPW, RPW)],
                    out_hbm.at[c, 1, pl.ds(s * RPW, RPW)])


@functools.cache
def _deg_call():
    return pl.kernel(
        _deg_body,
        out_type=jax.ShapeDtypeStruct((NCORE, 2, NPAD, 16), jnp.float32),
        mesh=_mesh(),
        scratch_types=[
            pltpu.VMEM((NCHUNK, CHUNK), jnp.int32),
            pltpu.VMEM((NCHUNK, CHUNK), jnp.int32),
            pltpu.VMEM((CHUNK, 16), jnp.float32),
            pltpu.VMEM_SHARED((NPAD, 16), jnp.float32),
            pltpu.VMEM_SHARED((NPAD, 16), jnp.float32),
        ],
        compiler_params=pltpu.CompilerParams(use_tc_tiling_on_sc=False),
    )


def _seg_body(srcg_hbm, dst_hbm, table_hbm, zeros_hbm, out_hbm,
              idxs, idxd, rows0, acc, sem0):
    c = lax.axis_index("c")
    s = lax.axis_index("s")
    w = s * NCORE + c
    pltpu.sync_copy(srcg_hbm.at[w], idxs)
    pltpu.sync_copy(dst_hbm.at[w], idxd)
    pltpu.sync_copy(zeros_hbm, acc.at[pl.ds(s * RPW, RPW)])
    plsc.subcore_barrier()

    def body(j, carry):
        pltpu.async_copy(table_hbm.at[idxs.at[j]], rows0, sem0).wait()
        pltpu.sync_copy(rows0, acc.at[idxd.at[j]], add=True)
        return carry

    lax.fori_loop(0, NCHUNK, body, 0)
    plsc.subcore_barrier()
    pltpu.sync_copy(acc.at[pl.ds(s * RPW, RPW)],
                    out_hbm.at[c, pl.ds(s * RPW, RPW)])


@functools.cache
def _seg_call(d):
    return pl.kernel(
        _seg_body,
        out_type=jax.ShapeDtypeStruct((NCORE, NPAD, d), jnp.float32),
        mesh=_mesh(),
        scratch_types=[
            pltpu.VMEM((NCHUNK, CHUNK), jnp.int32),
            pltpu.VMEM((NCHUNK, CHUNK), jnp.int32),
            pltpu.VMEM((CHUNK, d), jnp.float32),
            pltpu.VMEM_SHARED((NPAD, d), jnp.float32),
            pltpu.SemaphoreType.DMA,
        ],
        compiler_params=pltpu.CompilerParams(use_tc_tiling_on_sc=False),
    )


def _norm_from(deg2):
    return lax.rsqrt(jnp.maximum(deg2, 1.0))[:, 0:1]


def _mm1_body(x_ref, w1_ref, out_ref):
    w1 = w1_ref[...]
    parts = [jnp.dot(x_ref[t], w1, preferred_element_type=jnp.float32)
             for t in range(T)]
    out_ref[...] = jnp.concatenate(parts, axis=1)


def _scale_body(p_ref, deg_ref, out_ref):
    no = _norm_from(deg_ref[0, 0] + deg_ref[1, 0])
    out_ref[...] = p_ref[...] * no


def _mm2_body(m1_ref, deg_ref, b1_ref, w2_ref, out_ref):
    no = _norm_from(deg_ref[0, 0] + deg_ref[1, 0])
    ni = _norm_from(deg_ref[0, 1] + deg_ref[1, 1])
    m1 = m1_ref[0] + m1_ref[1]
    h1 = jnp.maximum(m1 * ni + b1_ref[...], 0.0) * no
    out_ref[...] = jnp.dot(h1, w2_ref[...], preferred_element_type=jnp.float32)


def _lstm_body(m2_ref, deg_ref, b2_ref, wih_ref, whh_ref, bg_ref,
               wout_ref, bo_ref, pool_ref, out_ref):
    ni = _norm_from(deg_ref[0, 1] + deg_ref[1, 1])
    m2 = m2_ref[0] + m2_ref[1]
    h2 = jnp.maximum(m2 * ni + b2_ref[...], 0.0)
    wih = wih_ref[...]
    whh = whh_ref[...]
    bg = bg_ref[...]
    h = jnp.zeros((BLK, 8), jnp.float32)
    c = jnp.zeros((BLK, 8), jnp.float32)
    for t in range(T):
        xt = h2[:, H2 * t:H2 * t + H2]
        g = (jnp.dot(xt, wih, preferred_element_type=jnp.float32)
             + jnp.dot(h, whh, preferred_element_type=jnp.float32) + bg)
        i = jax.nn.sigmoid(g[:, 0:8])
        f = jax.nn.sigmoid(g[:, 8:16])
        gg = jnp.tanh(g[:, 16:24])
        o = jax.nn.sigmoid(g[:, 24:32])
        c = f * c + i * gg
        h = o * jnp.tanh(c)
    bmax = jnp.max(h, axis=0, keepdims=True)

    @pl.when(pl.program_id(0) == 0)
    def _():
        pool_ref[...] = bmax

    @pl.when(pl.program_id(0) > 0)
    def _():
        pool_ref[...] = jnp.maximum(pool_ref[...], bmax)

    @pl.when(pl.program_id(0) == NBLK - 1)
    def _():
        out_ref[...] = jax.nn.sigmoid(
            jnp.dot(pool_ref[...], wout_ref[...],
                    preferred_element_type=jnp.float32) + bo_ref[...])


_DEG_SPEC = pl.BlockSpec((NCORE, 2, BLK, 16), lambda n: (0, 0, n, 0))


_mm1_call = pl.pallas_call(
    _mm1_body,
    grid=(NBLK,),
    in_specs=[
        pl.BlockSpec((T, BLK, F_IN), lambda n: (0, n, 0)),
        pl.BlockSpec((F_IN, H1), lambda n: (0, 0)),
    ],
    out_specs=pl.BlockSpec((BLK, D1), lambda n: (n, 0)),
    out_shape=jax.ShapeDtypeStruct((N, D1), jnp.float32),
)

_scale_call = pl.pallas_call(
    _scale_body,
    grid=(NBLK,),
    in_specs=[
        pl.BlockSpec((BLK, D1), lambda n: (n, 0)),
        _DEG_SPEC,
    ],
    out_specs=pl.BlockSpec((BLK, D1), lambda n: (n, 0)),
    out_shape=jax.ShapeDtypeStruct((N, D1), jnp.float32),
)

_mm2_call = pl.pallas_call(
    _mm2_body,
    grid=(NBLK,),
    in_specs=[
        pl.BlockSpec((NCORE, BLK, D1), lambda n: (0, n, 0)),
        _DEG_SPEC,
        pl.BlockSpec((1, D1), lambda n: (0, 0)),
        pl.BlockSpec((D1, D2), lambda n: (0, 0)),
    ],
    out_specs=pl.BlockSpec((BLK, D2), lambda n: (n, 0)),
    out_shape=jax.ShapeDtypeStruct((N, D2), jnp.float32),
)

_lstm_call = pl.pallas_call(
    _lstm_body,
    grid=(NBLK,),
    in_specs=[
        pl.BlockSpec((NCORE, BLK, D2), lambda n: (0, n, 0)),
        _DEG_SPEC,
        pl.BlockSpec((1, D2), lambda n: (0, 0)),
        pl.BlockSpec((H2, 32), lambda n: (0, 0)),
        pl.BlockSpec((8, 32), lambda n: (0, 0)),
        pl.BlockSpec((1, 32), lambda n: (0, 0)),
        pl.BlockSpec((8, 4), lambda n: (0, 0)),
        pl.BlockSpec((1, 4), lambda n: (0, 0)),
    ],
    out_specs=[
        pl.BlockSpec((1, 8), lambda n: (0, 0)),
        pl.BlockSpec((1, 4), lambda n: (0, 0)),
    ],
    out_shape=[
        jax.ShapeDtypeStruct((1, 8), jnp.float32),
        jax.ShapeDtypeStruct((1, 4), jnp.float32),
    ],
)


def _prep_indices(edge_index):
    src = edge_index[0]
    dst = edge_index[1]
    e = src.shape[0]
    pad = NW * EPW - e
    k = jnp.arange(pad, dtype=jnp.int32)
    trash = N + k % (NPAD - N)
    srcg = jnp.concatenate([src, k % 4096])
    srcd = jnp.concatenate([src, trash])
    dst3 = jnp.concatenate([dst, trash])
    srcg = srcg.reshape(NW, NCHUNK, CHUNK)
    srcd = srcd.reshape(NW, NCHUNK, CHUNK)
    dst3 = dst3.reshape(NW, NCHUNK, CHUNK)
    return srcg, srcd, dst3


@jax.jit
def kernel(x, edge_index, W1, b1, W2, b2, W_ih, W_hh, b_ih, b_hh,
           W_out, b_out):
    srcg, srcd, dst3 = _prep_indices(edge_index)

    ones16 = jnp.ones((CHUNK, 16), jnp.float32)
    zeros16 = jnp.zeros((RPW, 16), jnp.float32)
    zeros64 = jnp.zeros((RPW, D1), jnp.float32)
    zeros128 = jnp.zeros((RPW, D2), jnp.float32)

    degp = _deg_call()(srcd, dst3, ones16, zeros16)
    p1u = _mm1_call(x, W1)
    p1s = _scale_call(p1u, degp)
    m1p = _seg_call(D1)(srcg, dst3, p1s, zeros64)
    w2big = jnp.kron(jnp.eye(T, dtype=jnp.float32), W2)
    p2s = _mm2_call(m1p, degp, jnp.tile(b1, T)[None], w2big)
    m2p = _seg_call(D2)(srcg, dst3, p2s, zeros128)
    _, out = _lstm_call(m2p, degp, jnp.tile(b2, T)[None],
                        W_ih.T, W_hh.T, (b_ih + b_hh)[None],
                        W_out.T, b_out[None])
    return out
